# Initial kernel scaffold; baseline (speedup 1.0000x reference)
import jax
import jax.numpy as jnp
from jax.experimental import pallas as pl


def kernel(pos, features, W_emb, b_emb, W_q, W_k, W_v, Wp1, bp1, Wp2, bp2, Wa1, ba1, Wa2, ba2, W_out, b_out):
    raise NotImplementedError("write your pallas kernel here")


# speedup vs baseline: 13.7062x; 13.7062x over previous
import functools

import jax
import jax.numpy as jnp
from jax import lax
from jax.experimental import pallas as pl
from jax.experimental.pallas import tpu as pltpu

B, N, C, K = 4, 1024, 256, 16
QB = 128
NQB = N // QB

_f32 = jnp.float32


def _prep_body(pos_ref, posT_ref, feat_ref, W_emb_ref, b_emb_ref,
               W_q_ref, W_k_ref, W_v_ref,
               q_ref, xk_ref, xv_ref, knn_ref):
    pos = pos_ref[0]
    posT = posT_ref[0]
    x = jnp.dot(feat_ref[0], W_emb_ref[...],
                preferred_element_type=_f32) + b_emb_ref[...]
    q_ref[0] = jnp.dot(x, W_q_ref[...], preferred_element_type=_f32)
    xk_ref[0] = jnp.dot(x, W_k_ref[...], preferred_element_type=_f32)
    xv_ref[0] = jnp.dot(x, W_v_ref[...], preferred_element_type=_f32)

    d2 = (pos[:, 0:1] - posT[0:1, :]) ** 2
    d2 = d2 + (pos[:, 1:2] - posT[1:2, :]) ** 2
    d2 = d2 + (pos[:, 2:3] - posT[2:3, :]) ** 2

    iota = lax.broadcasted_iota(jnp.int32, (N, N), 1)
    work = d2
    cols = []
    for _ in range(K):
        m = jnp.min(work, axis=1, keepdims=True)
        idx = jnp.min(jnp.where(work == m, iota, N), axis=1, keepdims=True)
        cols.append(idx)
        work = jnp.where(iota == idx, jnp.inf, work)
    knn_ref[0] = jnp.concatenate(cols, axis=1)


def _attn_body(q_ref, xk_ref, xv_ref, pos_ref, posq_ref, knn_ref, feat_ref,
               Wp1_ref, bp1_ref, Wp2_ref, bp2_ref,
               Wa1_ref, ba1_ref, Wa2_ref, ba2_ref,
               W_out_ref, b_out_ref, out_ref):
    knn = knn_ref[0]
    iota3 = lax.broadcasted_iota(jnp.int32, (QB, K, N), 2)
    onehot = jnp.where(knn[:, :, None] == iota3, 1.0, 0.0).astype(_f32)
    onehot = onehot.reshape(QB * K, N)

    kk = jnp.dot(onehot, xk_ref[0], preferred_element_type=_f32)
    v = jnp.dot(onehot, xv_ref[0], preferred_element_type=_f32)
    gpos = jnp.dot(onehot, pos_ref[0], preferred_element_type=_f32)

    posq = posq_ref[0]
    posq_rep = jnp.broadcast_to(posq[:, None, :], (QB, K, 3)).reshape(QB * K, 3)
    gpos = gpos - posq_rep

    h = jnp.maximum(
        jnp.dot(gpos, Wp1_ref[...], preferred_element_type=_f32) + bp1_ref[...],
        0.0)
    posenc = jnp.dot(h, Wp2_ref[...], preferred_element_type=_f32) + bp2_ref[...]

    q = q_ref[0]
    q_rep = jnp.broadcast_to(q[:, None, :], (QB, K, C)).reshape(QB * K, C)
    pre = q_rep - kk + posenc
    h2 = jnp.maximum(
        jnp.dot(pre, Wa1_ref[...], preferred_element_type=_f32) + ba1_ref[...],
        0.0)
    attn = jnp.dot(h2, Wa2_ref[...], preferred_element_type=_f32) + ba2_ref[...]

    s = (attn * (1.0 / 16.0)).reshape(QB, K, C)
    m = jnp.max(s, axis=1, keepdims=True)
    e = jnp.exp(s - m)
    w = e / jnp.sum(e, axis=1, keepdims=True)

    vp = (v + posenc).reshape(QB, K, C)
    res = jnp.sum(w * vp, axis=1)
    out_ref[0] = (jnp.dot(res, W_out_ref[...], preferred_element_type=_f32)
                  + b_out_ref[...] + feat_ref[0])


def _full(shape):
    return pl.BlockSpec(shape, lambda *args: tuple(0 for _ in shape))


def kernel(pos, features, W_emb, b_emb, W_q, W_k, W_v, Wp1, bp1, Wp2, bp2,
           Wa1, ba1, Wa2, ba2, W_out, b_out, *, interpret=False):
    posT = jnp.transpose(pos, (0, 2, 1))
    b_emb2 = b_emb.reshape(1, C)
    bp12 = bp1.reshape(1, C)
    bp22 = bp2.reshape(1, C)
    ba12 = ba1.reshape(1, C)
    ba22 = ba2.reshape(1, C)
    b_out2 = b_out.reshape(1, -1)

    q, xk, xv, knn = pl.pallas_call(
        _prep_body,
        grid=(B,),
        in_specs=[
            pl.BlockSpec((1, N, 3), lambda b: (b, 0, 0)),
            pl.BlockSpec((1, 3, N), lambda b: (b, 0, 0)),
            pl.BlockSpec((1, N, C), lambda b: (b, 0, 0)),
            _full(W_emb.shape), _full(b_emb2.shape),
            _full(W_q.shape), _full(W_k.shape), _full(W_v.shape),
        ],
        out_specs=[
            pl.BlockSpec((1, N, C), lambda b: (b, 0, 0)),
            pl.BlockSpec((1, N, C), lambda b: (b, 0, 0)),
            pl.BlockSpec((1, N, C), lambda b: (b, 0, 0)),
            pl.BlockSpec((1, N, K), lambda b: (b, 0, 0)),
        ],
        out_shape=[
            jax.ShapeDtypeStruct((B, N, C), _f32),
            jax.ShapeDtypeStruct((B, N, C), _f32),
            jax.ShapeDtypeStruct((B, N, C), _f32),
            jax.ShapeDtypeStruct((B, N, K), jnp.int32),
        ],
        interpret=interpret,
    )(pos, posT, features, W_emb, b_emb2, W_q, W_k, W_v)

    out = pl.pallas_call(
        _attn_body,
        grid=(B, NQB),
        in_specs=[
            pl.BlockSpec((1, QB, C), lambda b, qb: (b, qb, 0)),
            pl.BlockSpec((1, N, C), lambda b, qb: (b, 0, 0)),
            pl.BlockSpec((1, N, C), lambda b, qb: (b, 0, 0)),
            pl.BlockSpec((1, N, 3), lambda b, qb: (b, 0, 0)),
            pl.BlockSpec((1, QB, 3), lambda b, qb: (b, qb, 0)),
            pl.BlockSpec((1, QB, K), lambda b, qb: (b, qb, 0)),
            pl.BlockSpec((1, QB, C), lambda b, qb: (b, qb, 0)),
            _full(Wp1.shape), _full(bp12.shape),
            _full(Wp2.shape), _full(bp22.shape),
            _full(Wa1.shape), _full(ba12.shape),
            _full(Wa2.shape), _full(ba22.shape),
            _full(W_out.shape), _full(b_out2.shape),
        ],
        out_specs=pl.BlockSpec((1, QB, C), lambda b, qb: (b, qb, 0)),
        out_shape=jax.ShapeDtypeStruct((B, N, C), _f32),
        interpret=interpret,
    )(q, xk, xv, pos, pos, knn, features,
      Wp1, bp12, Wp2, bp22, Wa1, ba12, Wa2, ba22, W_out, b_out2)
    return out
